# Initial kernel scaffold; baseline (speedup 1.0000x reference)
import jax
import jax.numpy as jnp
from jax.experimental import pallas as pl


def kernel(seed_index, es, ps, ep_adj_indices, ep_adj_values, pe_adj_indices, pe_adj_values, We, Wp, Wg):
    raise NotImplementedError("write your pallas kernel here")


# speedup vs baseline: 3.7326x; 3.7326x over previous
import functools

import jax
import jax.numpy as jnp
from jax import lax
from jax.experimental import pallas as pl
from jax.experimental.pallas import tpu as pltpu
from jax.experimental.pallas import tpu_sc as plsc

N = 10000
D = 128
E_TOTAL = 320000
NSEED = 200
NC, NS = 2, 16
EPT = E_TOTAL // (NC * NS)
CHUNK = 80
NCHUNK = EPT // CHUNK
N_PAD = 10240
RPT = N_PAD // NS


def _sc_body(do_seeds, x_hbm, cols_hbm, dsts_hbm, vals_hbm, zeros_hbm,
             seed_hbm, acc_hbm, g_hbm,
             col_v, dst_v, val_v, rows_v, sidx_v, seeds_v, gsum_v, acc_s, sem):
    c = lax.axis_index("c")
    s = lax.axis_index("s")
    wid = c * NS + s

    pltpu.sync_copy(zeros_hbm, acc_s.at[pl.ds(s * RPT, RPT)])
    plsc.subcore_barrier()

    base_e = wid * EPT

    def chunk_body(k, carry):
        eb = base_e + k * CHUNK
        pltpu.sync_copy(cols_hbm.at[pl.ds(eb, CHUNK)], col_v)
        pltpu.sync_copy(dsts_hbm.at[pl.ds(eb, CHUNK)], dst_v)
        pltpu.sync_copy(vals_hbm.at[pl.ds(eb, CHUNK)], val_v)
        pltpu.async_copy(x_hbm.at[col_v], rows_v, sem).wait()

        gdn = lax.GatherDimensionNumbers(offset_dims=(),
                                         collapsed_slice_dims=(0,),
                                         start_index_map=(0,))

        def scale_group(gi, carry2):
            v16 = val_v[pl.ds(gi * 16, 16)]
            for rl in range(16):
                bv = lax.gather(v16, jnp.full((16, 1), rl, jnp.int32),
                                gdn, (1,),
                                mode=lax.GatherScatterMode.PROMISE_IN_BOUNDS)
                r = gi * 16 + rl
                for j in range(D // 16):
                    sl = pl.ds(j * 16, 16)
                    rows_v[r, sl] = rows_v[r, sl] * bv
            return carry2

        lax.fori_loop(0, CHUNK // 16, scale_group, 0, unroll=False)
        pltpu.sync_copy(rows_v, acc_s.at[dst_v], add=True)
        return carry

    lax.fori_loop(0, NCHUNK, chunk_body, 0, unroll=False)
    plsc.subcore_barrier()

    pltpu.sync_copy(acc_s.at[pl.ds(s * RPT, RPT)],
                    acc_hbm.at[c, pl.ds(s * RPT, RPT)])

    if do_seeds:
        @pl.when(jnp.logical_and(c == 0, s == 0))
        def _():
            pltpu.sync_copy(seed_hbm, sidx_v)
            h = 104
            pltpu.async_copy(x_hbm.at[sidx_v.at[pl.ds(0, h)]],
                             seeds_v.at[pl.ds(0, h)], sem).wait()
            pltpu.async_copy(x_hbm.at[sidx_v.at[pl.ds(h, NSEED - h)]],
                             seeds_v.at[pl.ds(h, NSEED - h)], sem).wait()

            def seed_row(r, carry):
                return tuple(carry[j] + seeds_v[r, pl.ds(j * 16, 16)]
                             for j in range(D // 16))

            g = lax.fori_loop(0, NSEED, seed_row,
                              tuple(jnp.zeros((16,), jnp.float32)
                                    for _ in range(D // 16)))
            for j in range(D // 16):
                gsum_v[pl.ds(j * 16, 16)] = g[j]
            pltpu.sync_copy(gsum_v, g_hbm.at[0])


def _make_sc_spmm(do_seeds):
    scratch = [
        pltpu.VMEM((CHUNK,), jnp.int32),
        pltpu.VMEM((CHUNK,), jnp.int32),
        pltpu.VMEM((CHUNK,), jnp.float32),
        pltpu.VMEM((CHUNK, D), jnp.float32),
        pltpu.VMEM((NSEED,), jnp.int32),
        pltpu.VMEM((NSEED, D), jnp.float32),
        pltpu.VMEM((D,), jnp.float32),
        pltpu.VMEM_SHARED((N_PAD, D), jnp.float32),
        pltpu.SemaphoreType.DMA,
    ]
    out_type = [jax.ShapeDtypeStruct((NC, N_PAD, D), jnp.float32),
                jax.ShapeDtypeStruct((8, D), jnp.float32)]
    return pl.kernel(
        functools.partial(_sc_body, do_seeds),
        mesh=plsc.VectorSubcoreMesh(core_axis_name="c", subcore_axis_name="s"),
        out_type=out_type,
        scratch_types=scratch,
    )


def _dense_body(use_g, acc_ref, w_ref, res_ref, g_ref, wg_ref, out_ref):
    m = acc_ref[0] + acc_ref[1]
    y = jnp.dot(m, w_ref[...], preferred_element_type=jnp.float32)
    y = y + res_ref[...]
    if use_g:
        gw = jnp.dot(g_ref[0:1] * (1.0 / NSEED), wg_ref[...],
                     preferred_element_type=jnp.float32)
        y = y + gw
    out_ref[...] = jnp.maximum(y, 0.0)


def _tc_dense(acc, w, res, g, wg, use_g):
    blk = 2000
    grid = N // blk
    return pl.pallas_call(
        functools.partial(_dense_body, use_g),
        grid=(grid,),
        in_specs=[
            pl.BlockSpec((NC, blk, D), lambda i: (0, i, 0)),
            pl.BlockSpec((D, D), lambda i: (0, 0)),
            pl.BlockSpec((blk, D), lambda i: (i, 0)),
            pl.BlockSpec((8, D), lambda i: (0, 0)),
            pl.BlockSpec((D, D), lambda i: (0, 0)),
        ],
        out_specs=pl.BlockSpec((blk, D), lambda i: (i, 0)),
        out_shape=jax.ShapeDtypeStruct((N, D), jnp.float32),
    )(acc, w, res, g, wg)


def kernel(seed_index, es, ps, ep_adj_indices, ep_adj_values,
           pe_adj_indices, pe_adj_values, We, Wp, Wg):
    si = seed_index.astype(jnp.int32)
    pe0 = pe_adj_indices[0].astype(jnp.int32)
    pe1 = pe_adj_indices[1].astype(jnp.int32)
    ep0 = ep_adj_indices[0].astype(jnp.int32)
    ep1 = ep_adj_indices[1].astype(jnp.int32)
    zeros = jnp.zeros((RPT, D), jnp.float32)
    sc_seed = _make_sc_spmm(True)
    sc_plain = _make_sc_spmm(False)
    L = We.shape[0]
    for i in range(L):
        acc_p, gsum = sc_seed(es, pe1, pe0, pe_adj_values, zeros, si)
        ps = _tc_dense(acc_p, Wp[i], ps, gsum, Wg[i], False)
        acc_e, _ = sc_plain(ps, ep1, ep0, ep_adj_values, zeros, si)
        es = _tc_dense(acc_e, We[i], es, gsum, Wg[i], True)
    return es, ps


# speedup vs baseline: 11.2337x; 3.0096x over previous
import functools

import jax
import jax.numpy as jnp
from jax import lax
from jax.experimental import pallas as pl
from jax.experimental.pallas import tpu as pltpu
from jax.experimental.pallas import tpu_sc as plsc

N = 10000
D = 128
E_TOTAL = 320000
NSEED = 200
NC, NS = 2, 16
EPT = E_TOTAL // (NC * NS)
CHUNK = 80
NCHUNK = EPT // CHUNK
N_PAD = 10240
RPT = N_PAD // NS
NBUF = 4
NSLOT = ((NCHUNK + NBUF - 1) // NBUF) * NBUF


def _issue_idx(cols_hbm, dsts_hbm, vals_hbm, col_r, dst_r, val_r, sems,
               base_e, k, b):
    eb = base_e + k * CHUNK
    pltpu.async_copy(cols_hbm.at[pl.ds(eb, CHUNK)], col_r.at[b], sems[b])
    pltpu.async_copy(dsts_hbm.at[pl.ds(eb, CHUNK)], dst_r.at[b], sems[b])
    pltpu.async_copy(vals_hbm.at[pl.ds(eb, CHUNK)], val_r.at[b], sems[b])


def _wait_idx(cols_hbm, dsts_hbm, vals_hbm, col_r, dst_r, val_r, sems,
              base_e, k, b):
    eb = base_e + k * CHUNK
    pltpu.make_async_copy(cols_hbm.at[pl.ds(eb, CHUNK)], col_r.at[b],
                          sems[b]).wait()
    pltpu.make_async_copy(dsts_hbm.at[pl.ds(eb, CHUNK)], dst_r.at[b],
                          sems[b]).wait()
    pltpu.make_async_copy(vals_hbm.at[pl.ds(eb, CHUNK)], val_r.at[b],
                          sems[b]).wait()


def _sc_body(do_seeds, x_hbm, cols_hbm, dsts_hbm, vals_hbm, zeros_hbm,
             seed_hbm, acc_hbm, g_hbm,
             col_r, dst_r, val_r, rows, sidx_v, gsum_v, acc_s, isems, gsems):
    c = lax.axis_index("c")
    s = lax.axis_index("s")
    wid = c * NS + s

    pltpu.sync_copy(zeros_hbm, acc_s.at[pl.ds(s * RPT, RPT)])
    plsc.subcore_barrier()

    base_e = wid * EPT
    gdn = lax.GatherDimensionNumbers(offset_dims=(),
                                     collapsed_slice_dims=(0,),
                                     start_index_map=(0,))

    def scale_chunk(b):
        def scale_group(gi, carry2):
            v16 = val_r[b, pl.ds(gi * 16, 16)]
            buf = rows[b]
            for rl in range(16):
                bv = lax.gather(v16, jnp.full((16, 1), rl, jnp.int32),
                                gdn, (1,),
                                mode=lax.GatherScatterMode.PROMISE_IN_BOUNDS)
                for j in range(D // 16):
                    sl = pl.ds(j * 16, 16)
                    buf[gi * 16 + rl, sl] = buf[gi * 16 + rl, sl] * bv
            return carry2

        lax.fori_loop(0, CHUNK // 16, scale_group, 0, unroll=False)

    idx_args = (cols_hbm, dsts_hbm, vals_hbm, col_r, dst_r, val_r, isems,
                base_e)

    for kk in range(NBUF - 1):
        _issue_idx(*idx_args, kk, kk)
    for kk in range(NBUF - 2):
        _wait_idx(*idx_args, kk, kk)
        pltpu.async_copy(x_hbm.at[col_r.at[kk]], rows[kk], gsems[kk])

    def outer(o, carry):
        for b in range(NBUF):
            k = o * NBUF + b
            b_i = (b + NBUF - 1) % NBUF
            b_g = (b + NBUF - 2) % NBUF

            @pl.when(k + NBUF - 1 < NCHUNK)
            def _():
                _issue_idx(*idx_args, k + NBUF - 1, b_i)

            @pl.when(k + NBUF - 2 < NCHUNK)
            def _():
                _wait_idx(*idx_args, k + NBUF - 2, b_g)
                pltpu.async_copy(x_hbm.at[col_r.at[b_g]], rows[b_g],
                                 gsems[b_g])

            @pl.when(k < NCHUNK)
            def _():
                pltpu.make_async_copy(x_hbm.at[col_r.at[b]], rows[b],
                                      gsems[b]).wait()
                scale_chunk(b)
                pltpu.sync_copy(rows[b], acc_s.at[dst_r.at[b]], add=True)
        return carry

    lax.fori_loop(0, NSLOT // NBUF, outer, 0, unroll=False)
    plsc.subcore_barrier()

    pltpu.sync_copy(acc_s.at[pl.ds(s * RPT, RPT)],
                    acc_hbm.at[c, pl.ds(s * RPT, RPT)])

    if do_seeds:
        @pl.when(jnp.logical_and(c == 0, s == 0))
        def _():
            pltpu.sync_copy(seed_hbm, sidx_v)
            g = tuple(jnp.zeros((16,), jnp.float32) for _ in range(D // 16))
            for off, sz in ((0, CHUNK), (CHUNK, CHUNK), (2 * CHUNK,
                                                         NSEED - 2 * CHUNK)):
                pltpu.async_copy(x_hbm.at[sidx_v.at[pl.ds(off, sz)]],
                                 rows[0].at[pl.ds(0, sz)], gsems[0]).wait()

                def seed_row(r, carry):
                    return tuple(carry[j] + rows[0][r, pl.ds(j * 16, 16)]
                                 for j in range(D // 16))

                g = lax.fori_loop(0, sz, seed_row, g)
            for j in range(D // 16):
                gsum_v[pl.ds(j * 16, 16)] = g[j]
            pltpu.sync_copy(gsum_v, g_hbm.at[0])


def _make_sc_spmm(do_seeds):
    scratch = [
        pltpu.VMEM((NBUF, CHUNK), jnp.int32),
        pltpu.VMEM((NBUF, CHUNK), jnp.int32),
        pltpu.VMEM((NBUF, CHUNK), jnp.float32),
        [pltpu.VMEM((CHUNK, D), jnp.float32) for _ in range(NBUF)],
        pltpu.VMEM((NSEED,), jnp.int32),
        pltpu.VMEM((D,), jnp.float32),
        pltpu.VMEM_SHARED((N_PAD, D), jnp.float32),
        [pltpu.SemaphoreType.DMA for _ in range(NBUF)],
        [pltpu.SemaphoreType.DMA for _ in range(NBUF)],
    ]
    out_type = [jax.ShapeDtypeStruct((NC, N_PAD, D), jnp.float32),
                jax.ShapeDtypeStruct((8, D), jnp.float32)]
    return pl.kernel(
        functools.partial(_sc_body, do_seeds),
        mesh=plsc.VectorSubcoreMesh(core_axis_name="c", subcore_axis_name="s"),
        out_type=out_type,
        scratch_types=scratch,
    )


def _dense_body(use_g, acc_ref, w_ref, res_ref, g_ref, wg_ref, out_ref):
    m = acc_ref[0] + acc_ref[1]
    y = jnp.dot(m, w_ref[...], preferred_element_type=jnp.float32)
    y = y + res_ref[...]
    if use_g:
        gw = jnp.dot(g_ref[0:1] * (1.0 / NSEED), wg_ref[...],
                     preferred_element_type=jnp.float32)
        y = y + gw
    out_ref[...] = jnp.maximum(y, 0.0)


def _tc_dense(acc, w, res, g, wg, use_g):
    blk = 2000
    grid = N // blk
    return pl.pallas_call(
        functools.partial(_dense_body, use_g),
        grid=(grid,),
        in_specs=[
            pl.BlockSpec((NC, blk, D), lambda i: (0, i, 0)),
            pl.BlockSpec((D, D), lambda i: (0, 0)),
            pl.BlockSpec((blk, D), lambda i: (i, 0)),
            pl.BlockSpec((8, D), lambda i: (0, 0)),
            pl.BlockSpec((D, D), lambda i: (0, 0)),
        ],
        out_specs=pl.BlockSpec((blk, D), lambda i: (i, 0)),
        out_shape=jax.ShapeDtypeStruct((N, D), jnp.float32),
    )(acc, w, res, g, wg)


def kernel(seed_index, es, ps, ep_adj_indices, ep_adj_values,
           pe_adj_indices, pe_adj_values, We, Wp, Wg):
    si = seed_index.astype(jnp.int32)
    pe0 = pe_adj_indices[0].astype(jnp.int32)
    pe1 = pe_adj_indices[1].astype(jnp.int32)
    ep0 = ep_adj_indices[0].astype(jnp.int32)
    ep1 = ep_adj_indices[1].astype(jnp.int32)
    pe_v = pe_adj_values.astype(jnp.float32)
    ep_v = ep_adj_values.astype(jnp.float32)
    zeros = jnp.zeros((RPT, D), jnp.float32)
    sc_seed = _make_sc_spmm(True)
    sc_plain = _make_sc_spmm(False)
    L = We.shape[0]
    for i in range(L):
        acc_p, gsum = sc_seed(es, pe1, pe0, pe_v, zeros, si)
        ps = _tc_dense(acc_p, Wp[i], ps, gsum, Wg[i], False)
        acc_e, _ = sc_plain(ps, ep1, ep0, ep_v, zeros, si)
        es = _tc_dense(acc_e, We[i], es, gsum, Wg[i], True)
    return es, ps


# speedup vs baseline: 11.9345x; 1.0624x over previous
import functools

import jax
import jax.numpy as jnp
from jax import lax
from jax.experimental import pallas as pl
from jax.experimental.pallas import tpu as pltpu
from jax.experimental.pallas import tpu_sc as plsc

N = 10000
D = 128
E_TOTAL = 320000
NSEED = 200
NC, NS = 2, 16
EPT = E_TOTAL // (NC * NS)
CHUNK = 80
NCHUNK = EPT // CHUNK
N_PAD = 10240
RPT = N_PAD // NS
NBUF = 4
NSLOT = ((NCHUNK + NBUF - 1) // NBUF) * NBUF


def _issue_idx(cols_hbm, dsts_hbm, vals_hbm, col_r, dst_r, val_r, sems,
               base_e, k, b):
    eb = base_e + k * CHUNK
    pltpu.async_copy(cols_hbm.at[pl.ds(eb, CHUNK)], col_r.at[b], sems[b])
    pltpu.async_copy(dsts_hbm.at[pl.ds(eb, CHUNK)], dst_r.at[k % (2 * NBUF)],
                     sems[b])
    pltpu.async_copy(vals_hbm.at[pl.ds(eb, CHUNK)], val_r.at[b], sems[b])


def _wait_idx(cols_hbm, dsts_hbm, vals_hbm, col_r, dst_r, val_r, sems,
              base_e, k, b):
    eb = base_e + k * CHUNK
    pltpu.make_async_copy(cols_hbm.at[pl.ds(eb, CHUNK)], col_r.at[b],
                          sems[b]).wait()
    pltpu.make_async_copy(dsts_hbm.at[pl.ds(eb, CHUNK)],
                          dst_r.at[k % (2 * NBUF)], sems[b]).wait()
    pltpu.make_async_copy(vals_hbm.at[pl.ds(eb, CHUNK)], val_r.at[b],
                          sems[b]).wait()


def _sc_body(do_seeds, x_hbm, cols_hbm, dsts_hbm, vals_hbm, zeros_hbm,
             seed_hbm, acc_hbm, g_hbm,
             col_r, dst_r, val_r, rows, sidx_v, gsum_v, acc_s, isems, gsems,
             ssems):
    c = lax.axis_index("c")
    s = lax.axis_index("s")
    wid = c * NS + s
    base_e = wid * EPT
    gdn = lax.GatherDimensionNumbers(offset_dims=(),
                                     collapsed_slice_dims=(0,),
                                     start_index_map=(0,))

    def scale_chunk(b):
        def scale_group(gi, carry2):
            v16 = val_r[b, pl.ds(gi * 16, 16)]
            buf = rows[b]
            for rl in range(16):
                bv = lax.gather(v16, jnp.full((16, 1), rl, jnp.int32),
                                gdn, (1,),
                                mode=lax.GatherScatterMode.PROMISE_IN_BOUNDS)
                for j in range(D // 16):
                    sl = pl.ds(j * 16, 16)
                    buf[gi * 16 + rl, sl] = buf[gi * 16 + rl, sl] * bv
            return carry2

        lax.fori_loop(0, CHUNK // 16, scale_group, 0, unroll=False)

    idx_args = (cols_hbm, dsts_hbm, vals_hbm, col_r, dst_r, val_r, isems,
                base_e)

    for kk in range(NBUF - 1):
        _issue_idx(*idx_args, kk, kk)
    pltpu.sync_copy(zeros_hbm, acc_s.at[pl.ds(s * RPT, RPT)])
    for kk in range(NBUF - 2):
        _wait_idx(*idx_args, kk, kk)
        pltpu.async_copy(x_hbm.at[col_r.at[kk]], rows[kk], gsems[kk])
    plsc.subcore_barrier()

    def outer(o, carry):
        for b in range(NBUF):
            k = o * NBUF + b
            b_i = (b + NBUF - 1) % NBUF
            b_g = (b + NBUF - 2) % NBUF

            @pl.when(jnp.logical_and(k >= 2, k < NCHUNK + 2))
            def _():
                pltpu.make_async_copy(
                    rows[b_g], acc_s.at[dst_r.at[(k - 2) % (2 * NBUF)]],
                    ssems[b_g]).wait()

            @pl.when(k + NBUF - 1 < NCHUNK)
            def _():
                _issue_idx(*idx_args, k + NBUF - 1, b_i)

            @pl.when(k + NBUF - 2 < NCHUNK)
            def _():
                _wait_idx(*idx_args, k + NBUF - 2, b_g)
                pltpu.async_copy(x_hbm.at[col_r.at[b_g]], rows[b_g],
                                 gsems[b_g])

            @pl.when(k < NCHUNK)
            def _():
                pltpu.make_async_copy(x_hbm.at[col_r.at[b]], rows[b],
                                      gsems[b]).wait()
                scale_chunk(b)
                pltpu.async_copy(rows[b],
                                 acc_s.at[dst_r.at[k % (2 * NBUF)]],
                                 ssems[b], add=True)
        return carry

    lax.fori_loop(0, NSLOT // NBUF, outer, 0, unroll=False)
    plsc.subcore_barrier()

    pltpu.sync_copy(acc_s.at[pl.ds(s * RPT, RPT)],
                    acc_hbm.at[c, pl.ds(s * RPT, RPT)])

    if do_seeds:
        @pl.when(jnp.logical_and(c == 0, s == 0))
        def _():
            pltpu.sync_copy(seed_hbm, sidx_v)
            g = tuple(jnp.zeros((16,), jnp.float32) for _ in range(D // 16))
            for off, sz in ((0, CHUNK), (CHUNK, CHUNK), (2 * CHUNK,
                                                         NSEED - 2 * CHUNK)):
                pltpu.async_copy(x_hbm.at[sidx_v.at[pl.ds(off, sz)]],
                                 rows[0].at[pl.ds(0, sz)], gsems[0]).wait()

                def seed_row(r, carry):
                    return tuple(carry[j] + rows[0][r, pl.ds(j * 16, 16)]
                                 for j in range(D // 16))

                g = lax.fori_loop(0, sz, seed_row, g)
            for j in range(D // 16):
                gsum_v[pl.ds(j * 16, 16)] = g[j]
            pltpu.sync_copy(gsum_v, g_hbm.at[0])


def _make_sc_spmm(do_seeds):
    scratch = [
        pltpu.VMEM((NBUF, CHUNK), jnp.int32),
        pltpu.VMEM((2 * NBUF, CHUNK), jnp.int32),
        pltpu.VMEM((NBUF, CHUNK), jnp.float32),
        [pltpu.VMEM((CHUNK, D), jnp.float32) for _ in range(NBUF)],
        pltpu.VMEM((NSEED,), jnp.int32),
        pltpu.VMEM((D,), jnp.float32),
        pltpu.VMEM_SHARED((N_PAD, D), jnp.float32),
        [pltpu.SemaphoreType.DMA for _ in range(NBUF)],
        [pltpu.SemaphoreType.DMA for _ in range(NBUF)],
        [pltpu.SemaphoreType.DMA for _ in range(NBUF)],
    ]
    out_type = [jax.ShapeDtypeStruct((NC, N_PAD, D), jnp.float32),
                jax.ShapeDtypeStruct((8, D), jnp.float32)]
    return pl.kernel(
        functools.partial(_sc_body, do_seeds),
        mesh=plsc.VectorSubcoreMesh(core_axis_name="c", subcore_axis_name="s"),
        out_type=out_type,
        scratch_types=scratch,
    )


def _dense_body(use_g, acc_ref, w_ref, res_ref, g_ref, wg_ref, out_ref):
    m = acc_ref[0] + acc_ref[1]
    y = jnp.dot(m, w_ref[...], preferred_element_type=jnp.float32)
    y = y + res_ref[...]
    if use_g:
        gw = jnp.dot(g_ref[0:1] * (1.0 / NSEED), wg_ref[...],
                     preferred_element_type=jnp.float32)
        y = y + gw
    out_ref[...] = jnp.maximum(y, 0.0)


def _tc_dense(acc, w, res, g, wg, use_g):
    blk = 2000
    grid = N // blk
    return pl.pallas_call(
        functools.partial(_dense_body, use_g),
        grid=(grid,),
        in_specs=[
            pl.BlockSpec((NC, blk, D), lambda i: (0, i, 0)),
            pl.BlockSpec((D, D), lambda i: (0, 0)),
            pl.BlockSpec((blk, D), lambda i: (i, 0)),
            pl.BlockSpec((8, D), lambda i: (0, 0)),
            pl.BlockSpec((D, D), lambda i: (0, 0)),
        ],
        out_specs=pl.BlockSpec((blk, D), lambda i: (i, 0)),
        out_shape=jax.ShapeDtypeStruct((N, D), jnp.float32),
    )(acc, w, res, g, wg)


def kernel(seed_index, es, ps, ep_adj_indices, ep_adj_values,
           pe_adj_indices, pe_adj_values, We, Wp, Wg):
    si = seed_index.astype(jnp.int32)
    pe0 = pe_adj_indices[0].astype(jnp.int32)
    pe1 = pe_adj_indices[1].astype(jnp.int32)
    ep0 = ep_adj_indices[0].astype(jnp.int32)
    ep1 = ep_adj_indices[1].astype(jnp.int32)
    pe_v = pe_adj_values.astype(jnp.float32)
    ep_v = ep_adj_values.astype(jnp.float32)
    zeros = jnp.zeros((RPT, D), jnp.float32)
    sc_seed = _make_sc_spmm(True)
    sc_plain = _make_sc_spmm(False)
    L = We.shape[0]
    for i in range(L):
        acc_p, gsum = sc_seed(es, pe1, pe0, pe_v, zeros, si)
        ps = _tc_dense(acc_p, Wp[i], ps, gsum, Wg[i], False)
        acc_e, _ = sc_plain(ps, ep1, ep0, ep_v, zeros, si)
        es = _tc_dense(acc_e, We[i], es, gsum, Wg[i], True)
    return es, ps
